# baseline (speedup 1.0000x reference)
import functools

import jax
import jax.numpy as jnp
from jax import lax
from jax.experimental import pallas as pl
from jax.experimental.pallas import tpu as pltpu
from jax.experimental.pallas import tpu_sc as plsc

N = 10000
E = 320000
D = 128

NC = 2
NS = 16
NW = NC * NS
K = 128
ET = E + N
M = 2 * (-(-ET // (NW * K * 2)))
EP = NW * M * K
NP = 10112
RPT = NP // NS

_mesh = plsc.VectorSubcoreMesh(core_axis_name="c", subcore_axis_name="s")


@functools.partial(
    pl.kernel,
    mesh=_mesh,
    out_type=jax.ShapeDtypeStruct((NC, NP, D), jnp.float32),
    scratch_types=[
        pltpu.VMEM((M, K), jnp.int32),
        pltpu.VMEM((K, D), jnp.float32),
        pltpu.VMEM_SHARED((NP, D), jnp.float32),
        pltpu.SemaphoreType.DMA,
    ],
)
def _sc_degree(d_hbm, ones_hbm, z_hbm, out_hbm, d_v, ones_v, acc, ssem):
    c = lax.axis_index("c")
    t = lax.axis_index("s")
    w = t * NC + c
    pltpu.sync_copy(d_hbm.at[w], d_v)
    pltpu.sync_copy(ones_hbm, ones_v)
    pltpu.sync_copy(z_hbm, acc.at[pl.ds(t * RPT, RPT)])
    plsc.subcore_barrier()

    def body(i, carry):
        c0 = pltpu.async_copy(ones_v, acc.at[d_v.at[2 * i]], ssem, add=True)
        c1 = pltpu.async_copy(ones_v, acc.at[d_v.at[2 * i + 1]], ssem, add=True)
        c0.wait()
        c1.wait()
        return carry

    lax.fori_loop(0, M // 2, body, 0)
    plsc.subcore_barrier()
    pltpu.sync_copy(acc.at[pl.ds(t * RPT, RPT)], out_hbm.at[c, pl.ds(t * RPT, RPT)])


@functools.partial(
    pl.kernel,
    mesh=_mesh,
    out_type=jax.ShapeDtypeStruct((NC, NP, D), jnp.float32),
    scratch_types=[
        pltpu.VMEM((M, K), jnp.int32),
        pltpu.VMEM((K,), jnp.int32),
        pltpu.VMEM((K,), jnp.int32),
        pltpu.VMEM((K,), jnp.int32),
        pltpu.VMEM((K,), jnp.int32),
        pltpu.VMEM((K, D), jnp.float32),
        pltpu.VMEM((K, D), jnp.float32),
        pltpu.VMEM_SHARED((NP, D), jnp.float32),
        pltpu.SemaphoreType.DMA,
        pltpu.SemaphoreType.DMA,
        pltpu.SemaphoreType.DMA,
        pltpu.SemaphoreType.DMA,
    ],
)
def _sc_aggregate(g_hbm, sd_hbm, z_hbm, out_hbm,
                  sd_v, s_c0, s_c1, d_c0, d_c1, rows0, rows1, acc,
                  gsem0, gsem1, ssem0, ssem1):
    c = lax.axis_index("c")
    t = lax.axis_index("s")
    w = t * NC + c
    pltpu.sync_copy(sd_hbm.at[w], sd_v)
    pltpu.sync_copy(z_hbm, acc.at[pl.ds(t * RPT, RPT)])
    plsc.subcore_barrier()

    def unpack(j, s_c, d_c):
        row = sd_v.at[j]
        for l in range(K // 16):
            sl = pl.ds(l * 16, 16)
            v = row[sl]
            s_c[sl] = lax.bitwise_and(v, 0xFFFF)
            d_c[sl] = lax.shift_right_logical(v, 16)

    def body(j, carry):
        unpack(j, s_c0, d_c0)
        pltpu.async_copy(g_hbm.at[s_c0], rows0, gsem0).wait()
        pltpu.sync_copy(rows0, acc.at[d_c0], add=True)
        return carry

    lax.fori_loop(0, M, body, 0)
    plsc.subcore_barrier()
    pltpu.sync_copy(acc.at[pl.ds(t * RPT, RPT)], out_hbm.at[c, pl.ds(t * RPT, RPT)])


def _tc1_body(x_ref, degp_ref, w1_ref, t1_ref, invs_ref):
    deg = degp_ref[0][:, 0:1] + degp_ref[1][:, 0:1]
    invs = lax.rsqrt(jnp.maximum(deg, 1.0))
    invs_ref[...] = invs
    t1_ref[...] = jnp.dot(x_ref[...], w1_ref[...],
                          preferred_element_type=jnp.float32) * invs


def _tc_mid_body(p_ref, invs_ref, b_ref, w_ref, t_ref):
    agg = p_ref[0] + p_ref[1]
    invs = invs_ref[...]
    h = jax.nn.relu(agg * invs + b_ref[...])
    rows = lax.broadcasted_iota(jnp.int32, (NP, 1), 0)
    h = jnp.where(rows < N, h, 0.0)
    t_ref[...] = jnp.dot(h, w_ref[...], preferred_element_type=jnp.float32) * invs


def _tc_final_body(p_ref, invs_ref, b3_ref, wd_ref, bd_ref, emb_ref, rec_ref):
    emb = (p_ref[0] + p_ref[1]) * invs_ref[...] + b3_ref[...]
    emb_ref[...] = emb
    rec_ref[...] = jax.nn.sigmoid(
        jnp.dot(emb, wd_ref[...], preferred_element_type=jnp.float32) + bd_ref[...])


def kernel(x, edge_index, W1, b1, W2, b2, W3, b3, Wd, bd):
    pad = EP - ET
    loop = jnp.arange(N, dtype=jnp.int32)
    pad_s = jnp.zeros((pad,), jnp.int32)
    pad_d = N + (jnp.arange(pad, dtype=jnp.int32) % (NP - N))
    s_flat = jnp.concatenate([edge_index[0], loop, pad_s])
    d_flat = jnp.concatenate([edge_index[1], loop, pad_d])
    sd_arr = (s_flat | (d_flat << 16)).reshape(NW, M, K)
    d_arr = d_flat.reshape(NW, M, K)
    xp = jnp.pad(x, ((0, NP - N), (0, 0)))

    onesD = jnp.ones((K, D), jnp.float32)
    zD = jnp.zeros((RPT, D), jnp.float32)

    degp = _sc_degree(d_arr, onesD, zD)

    t1, invs = pl.pallas_call(
        _tc1_body,
        out_shape=(jax.ShapeDtypeStruct((NP, D), jnp.float32),
                   jax.ShapeDtypeStruct((NP, 1), jnp.float32)),
    )(xp, degp, W1)

    p1 = _sc_aggregate(t1, sd_arr, zD)
    t2 = pl.pallas_call(
        _tc_mid_body,
        out_shape=jax.ShapeDtypeStruct((NP, D), jnp.float32),
    )(p1, invs, b1, W2)

    p2 = _sc_aggregate(t2, sd_arr, zD)
    t3 = pl.pallas_call(
        _tc_mid_body,
        out_shape=jax.ShapeDtypeStruct((NP, D), jnp.float32),
    )(p2, invs, b2, W3)

    p3 = _sc_aggregate(t3, sd_arr, zD)
    emb, rec = pl.pallas_call(
        _tc_final_body,
        out_shape=(jax.ShapeDtypeStruct((NP, D), jnp.float32),
                   jax.ShapeDtypeStruct((NP, D), jnp.float32)),
    )(p3, invs, b3, Wd, bd)

    return emb[:N], rec[:N]


# speedup vs baseline: 1.6417x; 1.6417x over previous
import functools

import jax
import jax.numpy as jnp
from jax import lax
from jax.experimental import pallas as pl
from jax.experimental.pallas import tpu as pltpu
from jax.experimental.pallas import tpu_sc as plsc

N = 10000
E = 320000
D = 128

NC = 2
NS = 16
NW = NC * NS
K = 128
ET = E + N
M = -(-ET // (NW * K))
EP = NW * M * K
NP = 10112
RPT = NP // NS

_mesh = plsc.VectorSubcoreMesh(core_axis_name="c", subcore_axis_name="s")


@functools.partial(
    pl.kernel,
    mesh=_mesh,
    out_type=jax.ShapeDtypeStruct((NC, NP, D), jnp.float32),
    scratch_types=[
        pltpu.VMEM((M, K), jnp.int32),
        pltpu.VMEM((K, D), jnp.float32),
        pltpu.VMEM_SHARED((NP, D), jnp.float32),
    ],
)
def _sc_degree(d_hbm, ones_hbm, z_hbm, out_hbm, d_v, ones_v, acc):
    c = lax.axis_index("c")
    t = lax.axis_index("s")
    w = t * NC + c
    pltpu.sync_copy(d_hbm.at[w], d_v)
    pltpu.sync_copy(ones_hbm, ones_v)
    pltpu.sync_copy(z_hbm, acc.at[pl.ds(t * RPT, RPT)])
    plsc.subcore_barrier()

    def body(j, carry):
        pltpu.sync_copy(ones_v, acc.at[d_v.at[j]], add=True)
        return carry

    lax.fori_loop(0, M, body, 0)
    plsc.subcore_barrier()
    pltpu.sync_copy(acc.at[pl.ds(t * RPT, RPT)], out_hbm.at[c, pl.ds(t * RPT, RPT)])


@functools.partial(
    pl.kernel,
    mesh=_mesh,
    out_type=jax.ShapeDtypeStruct((NC, NP, D), jnp.float32),
    scratch_types=[
        pltpu.VMEM((M, K), jnp.int32),
        pltpu.VMEM((M, K), jnp.int32),
        pltpu.VMEM((K, D), jnp.float32),
        pltpu.VMEM_SHARED((NP, D), jnp.float32),
        pltpu.SemaphoreType.DMA,
    ],
)
def _sc_aggregate(g_hbm, s_hbm, d_hbm, z_hbm, out_hbm, s_v, d_v, rows_v, acc, sem):
    c = lax.axis_index("c")
    t = lax.axis_index("s")
    w = t * NC + c
    pltpu.sync_copy(s_hbm.at[w], s_v)
    pltpu.sync_copy(d_hbm.at[w], d_v)
    pltpu.sync_copy(z_hbm, acc.at[pl.ds(t * RPT, RPT)])
    plsc.subcore_barrier()

    def body(j, carry):
        pltpu.async_copy(g_hbm.at[s_v.at[j]], rows_v, sem).wait()
        pltpu.sync_copy(rows_v, acc.at[d_v.at[j]], add=True)
        return carry

    lax.fori_loop(0, M, body, 0)
    plsc.subcore_barrier()
    pltpu.sync_copy(acc.at[pl.ds(t * RPT, RPT)], out_hbm.at[c, pl.ds(t * RPT, RPT)])


def _tc1_body(x_ref, degp_ref, w1_ref, t1_ref, invs_ref):
    deg = degp_ref[0][:, 0:1] + degp_ref[1][:, 0:1]
    invs = lax.rsqrt(jnp.maximum(deg, 1.0))
    invs_ref[...] = invs
    t1_ref[...] = jnp.dot(x_ref[...], w1_ref[...],
                          preferred_element_type=jnp.float32) * invs


def _tc_mid_body(p_ref, invs_ref, b_ref, w_ref, t_ref):
    agg = p_ref[0] + p_ref[1]
    invs = invs_ref[...]
    h = jax.nn.relu(agg * invs + b_ref[...])
    rows = lax.broadcasted_iota(jnp.int32, (NP, 1), 0)
    h = jnp.where(rows < N, h, 0.0)
    t_ref[...] = jnp.dot(h, w_ref[...], preferred_element_type=jnp.float32) * invs


def _tc_final_body(p_ref, invs_ref, b3_ref, wd_ref, bd_ref, emb_ref, rec_ref):
    emb = (p_ref[0] + p_ref[1]) * invs_ref[...] + b3_ref[...]
    emb_ref[...] = emb
    rec_ref[...] = jax.nn.sigmoid(
        jnp.dot(emb, wd_ref[...], preferred_element_type=jnp.float32) + bd_ref[...])


def kernel(x, edge_index, W1, b1, W2, b2, W3, b3, Wd, bd):
    pad = EP - ET
    loop = jnp.arange(N, dtype=jnp.int32)
    padv = jnp.full((pad,), N, jnp.int32)
    s_arr = jnp.concatenate([edge_index[0], loop, padv]).reshape(NW, M, K)
    d_arr = jnp.concatenate([edge_index[1], loop, padv]).reshape(NW, M, K)
    xp = jnp.pad(x, ((0, NP - N), (0, 0)))

    onesD = jnp.ones((K, D), jnp.float32)
    zD = jnp.zeros((RPT, D), jnp.float32)

    degp = _sc_degree(d_arr, onesD, zD)

    t1, invs = pl.pallas_call(
        _tc1_body,
        out_shape=(jax.ShapeDtypeStruct((NP, D), jnp.float32),
                   jax.ShapeDtypeStruct((NP, 1), jnp.float32)),
    )(xp, degp, W1)

    p1 = _sc_aggregate(t1, s_arr, d_arr, zD)
    t2 = pl.pallas_call(
        _tc_mid_body,
        out_shape=jax.ShapeDtypeStruct((NP, D), jnp.float32),
    )(p1, invs, b1, W2)

    p2 = _sc_aggregate(t2, s_arr, d_arr, zD)
    t3 = pl.pallas_call(
        _tc_mid_body,
        out_shape=jax.ShapeDtypeStruct((NP, D), jnp.float32),
    )(p2, invs, b2, W3)

    p3 = _sc_aggregate(t3, s_arr, d_arr, zD)
    emb, rec = pl.pallas_call(
        _tc_final_body,
        out_shape=(jax.ShapeDtypeStruct((NP, D), jnp.float32),
                   jax.ShapeDtypeStruct((NP, D), jnp.float32)),
    )(p3, invs, b3, Wd, bd)

    return emb[:N], rec[:N]
